# baseline (speedup 1.0000x reference)
import functools
import numpy as np
import jax
import jax.numpy as jnp
from jax import lax
from jax.experimental import pallas as pl
from jax.experimental.pallas import tpu as pltpu
from jax.experimental.pallas import tpu_sc as plsc

N = 10000
E = 320000
G = 100
K = 30
HID = 32

NC = 2
NS = 16
NW = NC * NS
L = 16

EPW = 10240
EC = 128
NCHUNK = EPW // EC
RPT = N // NS
RPT8 = 632
NB = 5

_mesh = plsc.VectorSubcoreMesh(core_axis_name="c", subcore_axis_name="s")
_sc_params = pltpu.CompilerParams(needs_layout_passes=False,
                                  use_tc_tiling_on_sc=False)


def _wid():
    return lax.axis_index("s") * NC + lax.axis_index("c")


@functools.partial(
    pl.kernel, mesh=_mesh, compiler_params=_sc_params,
    out_type=(jax.ShapeDtypeStruct((NW, N), jnp.float32),
              jax.ShapeDtypeStruct((NW, EPW), jnp.int32)),
    scratch_types=[
        pltpu.VMEM((N,), jnp.float32),
        pltpu.VMEM((EPW,), jnp.int32),
        pltpu.VMEM((EPW,), jnp.int32),
        pltpu.VMEM((EPW,), jnp.int32),
    ],
)
def _prep_sc(src_hbm, dst_hbm, out_hbm, src2_hbm, deg_v, src_v, dst_v, s2_v):
    w = _wid()
    pltpu.sync_copy(src_hbm.at[w], src_v)
    pltpu.sync_copy(dst_hbm.at[w], dst_v)
    z16 = jnp.zeros((L,), jnp.float32)

    def zero_body(j, _):
        deg_v[pl.ds(j * L, L)] = z16
        return 0
    lax.fori_loop(0, N // L, zero_body, 0)

    def step(i, _):
        s16 = src_v[pl.ds(i * L, L)]
        d16 = dst_v[pl.ds(i * L, L)]
        loop = s16 == d16
        m16 = jnp.where(loop, 0.0, 1.0).astype(jnp.float32)
        s2_v[pl.ds(i * L, L)] = jnp.where(loop, jnp.int32(N), s16)
        plsc.addupdate_scatter(deg_v, [d16], m16)
        return 0
    lax.fori_loop(0, EPW // L, step, 0)
    pltpu.sync_copy(deg_v, out_hbm.at[w])
    pltpu.sync_copy(s2_v, src2_hbm.at[w])


@functools.partial(
    pl.kernel, mesh=_mesh, compiler_params=_sc_params,
    out_type=jax.ShapeDtypeStruct((NC, N, HID), jnp.float32),
    scratch_types=[
        pltpu.VMEM_SHARED((N, HID), jnp.float32),
        pltpu.VMEM((EPW,), jnp.int32),
        pltpu.VMEM((NCHUNK, EC), jnp.int32),
        pltpu.VMEM((NB, EC, HID), jnp.float32),
        [pltpu.SemaphoreType.DMA] * NB,
        [pltpu.SemaphoreType.DMA] * NB,
    ],
)
def _agg_sc(xw_hbm, src_hbm, dst_hbm, zeros_hbm, out_hbm,
            agg_sp, src_v, dst_v, gbuf, gsem, ssem):
    c = lax.axis_index("c")
    s = lax.axis_index("s")
    w = _wid()
    rb = jnp.minimum(s * RPT8, N - RPT8)
    dz = pltpu.async_copy(zeros_hbm.at[pl.ds(rb, RPT8)],
                          agg_sp.at[pl.ds(rb, RPT8)], gsem[0])
    d1 = pltpu.async_copy(src_hbm.at[w], src_v, gsem[1])
    d2 = pltpu.async_copy(dst_hbm.at[w], dst_v, gsem[2])
    dz.wait(); d1.wait(); d2.wait()
    plsc.subcore_barrier()

    for b in range(NB):
        pltpu.async_copy(xw_hbm.at[src_v.at[pl.ds(b * EC, EC)]],
                         gbuf.at[b], gsem[b])

    def chunk(g, _):
        for b in range(NB):
            gg = g * NB + b
            pltpu.make_async_copy(xw_hbm.at[src_v.at[pl.ds(0, EC)]],
                                  gbuf.at[b], gsem[b]).wait()
            pltpu.async_copy(gbuf.at[b], agg_sp.at[dst_v.at[gg]], ssem[b],
                             add=True)

            @pl.when(gg + NB < NCHUNK)
            def _():
                pltpu.make_async_copy(gbuf.at[b], agg_sp.at[dst_v.at[gg]],
                                      ssem[b]).wait()
                pltpu.async_copy(
                    xw_hbm.at[src_v.at[pl.ds((gg + NB) * EC, EC)]],
                    gbuf.at[b], gsem[b])
        return 0
    lax.fori_loop(0, NCHUNK // NB, chunk, 0)
    for b in range(NB):
        pltpu.make_async_copy(gbuf.at[b], agg_sp.at[dst_v.at[0]],
                              ssem[b]).wait()
    plsc.subcore_barrier()
    pltpu.sync_copy(agg_sp.at[pl.ds(rb, RPT8)],
                    out_hbm.at[c, pl.ds(rb, RPT8)])


@functools.partial(
    pl.kernel, mesh=_mesh, compiler_params=_sc_params,
    out_type=jax.ShapeDtypeStruct((NW, N), jnp.float32),
    scratch_types=[
        pltpu.VMEM((N + 8,), jnp.float32),
        pltpu.VMEM((N,), jnp.float32),
        pltpu.VMEM((EPW,), jnp.int32),
        pltpu.VMEM((EPW,), jnp.int32),
    ],
)
def _agg1ch_sc(xw_hbm, src_hbm, dst_hbm, out_hbm, xw_v, acc_v, src_v, dst_v):
    w = _wid()
    pltpu.sync_copy(xw_hbm, xw_v)
    pltpu.sync_copy(src_hbm.at[w], src_v)
    pltpu.sync_copy(dst_hbm.at[w], dst_v)
    z16 = jnp.zeros((L,), jnp.float32)

    def zero_body(j, _):
        acc_v[pl.ds(j * L, L)] = z16
        return 0
    lax.fori_loop(0, N // L, zero_body, 0)

    def step(i, _):
        s16 = src_v[pl.ds(i * L, L)]
        d16 = dst_v[pl.ds(i * L, L)]
        v16 = plsc.load_gather(xw_v, [s16])
        plsc.addupdate_scatter(acc_v, [d16], v16)
        return 0
    lax.fori_loop(0, EPW // L, step, 0)
    pltpu.sync_copy(acc_v, out_hbm.at[w])


GPW = 4
AW = G // GPW
SPW = GPW * K

_NEG = np.float32(-3.4e38)


def _iota():
    return lax.iota(jnp.int32, L)


def _lane_i32(v16, lane):
    return jnp.max(jnp.where(_iota() == lane, v16, jnp.int32(-2**31)))


@functools.partial(
    pl.kernel, mesh=_mesh, compiler_params=_sc_params,
    out_type=(jax.ShapeDtypeStruct((G * K, HID), jnp.float32),
              jax.ShapeDtypeStruct((G * K, HID), jnp.float32),
              jax.ShapeDtypeStruct((G * K, HID), jnp.float32),
              jax.ShapeDtypeStruct((G * K,), jnp.float32)),
    scratch_types=[
        pltpu.VMEM((N,), jnp.float32),
        pltpu.VMEM((N,), jnp.int32),
        pltpu.VMEM((128,), jnp.int32),
        pltpu.VMEM((128,), jnp.int32),
        pltpu.VMEM((128,), jnp.int32),
        pltpu.VMEM((128,), jnp.float32),
        pltpu.VMEM((128, HID), jnp.float32),
        pltpu.VMEM((128, HID), jnp.float32),
        pltpu.VMEM((128, HID), jnp.float32),
        pltpu.SemaphoreType.DMA,
    ],
)
def _pool_sc(vals_hbm, batch_hbm, x1_hbm, x2_hbm, x3_hbm,
             o1_hbm, o2_hbm, o3_hbm, ov_hbm,
             vals_v, batch_v, cnt_v, starts_v, idx_v, valb_v,
             r1_v, r2_v, r3_v, sem):
    w = _wid()

    def body():
        pltpu.sync_copy(vals_hbm, vals_v)
        pltpu.sync_copy(batch_hbm, batch_v)
        z16i = jnp.zeros((L,), jnp.int32)
        z16f = jnp.zeros((L,), jnp.float32)
        one16 = jnp.ones((L,), jnp.int32)
        for j in range(128 // L):
            cnt_v[pl.ds(j * L, L)] = z16i
            idx_v[pl.ds(j * L, L)] = z16i
            valb_v[pl.ds(j * L, L)] = z16f

        def cnt_body(t, _):
            b16 = batch_v[pl.ds(t * L, L)]
            plsc.addupdate_scatter(cnt_v, [b16], one16)
            return 0
        lax.fori_loop(0, N // L, cnt_body, 0)

        carry = jnp.int32(0)
        for j in range(128 // L):
            c16 = cnt_v[pl.ds(j * L, L)]
            inc = plsc.cumsum(c16)
            starts_v[pl.ds(j * L, L)] = inc - c16 + carry
            carry = carry + jnp.sum(c16)

        for j in range(GPW):
            g = w * GPW + j
            gb = (g // L) * L
            s16 = starts_v[pl.ds(gb, L)]
            c16 = cnt_v[pl.ds(gb, L)]
            s = _lane_i32(s16, g - gb)
            c = _lane_i32(c16, g - gb)
            m = jnp.minimum(jnp.int32(K), c)
            b0 = (s // L) * L
            nch = (s + c - b0 + (L - 1)) // L

            def k_body(k, _):
                def t_body(t, bc):
                    bv, bi = bc
                    off = b0 + t * L
                    v = vals_v[pl.ds(off, L)]
                    gi = off + _iota()
                    ok = (gi >= s) & (gi < s + c)
                    vm = jnp.where(ok, v, _NEG)
                    cm = jnp.max(vm)
                    gmin = jnp.min(jnp.where(vm == cm, gi, jnp.int32(2**30)))
                    better = cm > bv
                    return (jnp.where(better, cm, bv),
                            jnp.where(better, gmin, bi))
                bv, bi = lax.fori_loop(0, nch, t_body,
                                       (jnp.float32(-2.0e38), jnp.int32(0)))
                slot16 = jnp.full((L,), j * K + k, jnp.int32)
                bi16 = jnp.full((L,), bi, jnp.int32)
                lane0 = _iota() == 0
                plsc.store_scatter(idx_v, [slot16], bi16, mask=lane0)
                plsc.store_scatter(valb_v, [slot16],
                                   jnp.full((L,), bv, jnp.float32), mask=lane0)
                plsc.store_scatter(vals_v, [bi16],
                                   jnp.full((L,), _NEG, jnp.float32), mask=lane0)
                return 0
            lax.fori_loop(0, m, k_body, 0)

        d1 = pltpu.async_copy(x1_hbm.at[idx_v], r1_v, sem)
        d2 = pltpu.async_copy(x2_hbm.at[idx_v], r2_v, sem)
        d3 = pltpu.async_copy(x3_hbm.at[idx_v], r3_v, sem)
        d1.wait(); d2.wait(); d3.wait()

        for j in range(GPW):
            g = w * GPW + j
            gb = (g // L) * L
            c16 = cnt_v[pl.ds(gb, L)]
            m = jnp.minimum(jnp.int32(K), _lane_i32(c16, g - gb))

            def z_body(k, _):
                slot16 = jnp.full((L,), j * K + k, jnp.int32)
                for rv in (r1_v, r2_v, r3_v):
                    plsc.store_scatter(rv, [slot16, _iota()], z16f)
                    plsc.store_scatter(rv, [slot16, _iota() + L], z16f)
                return 0
            lax.fori_loop(m, K, z_body, 0)

        pltpu.sync_copy(r1_v.at[pl.ds(0, SPW)], o1_hbm.at[pl.ds(w * SPW, SPW)])
        pltpu.sync_copy(r2_v.at[pl.ds(0, SPW)], o2_hbm.at[pl.ds(w * SPW, SPW)])
        pltpu.sync_copy(r3_v.at[pl.ds(0, SPW)], o3_hbm.at[pl.ds(w * SPW, SPW)])
        pltpu.sync_copy(valb_v.at[pl.ds(0, SPW)], ov_hbm.at[pl.ds(w * SPW, SPW)])

    pl.when(w < AW)(body)


def _prep_body(degp_ref, x_ref, w1_ref, dinv_ref, xw1_ref):
    deg = jnp.sum(degp_ref[...], axis=0) + 1.0
    dinv = lax.rsqrt(deg)[:, None]
    dinv_ref[...] = dinv
    xw1_ref[0:N, :] = (x_ref[...] @ w1_ref[...]) * dinv
    xw1_ref[N:N + 8, :] = jnp.zeros((8, HID), jnp.float32)


def _tc_prep(degp, x, w1):
    return pl.pallas_call(
        _prep_body,
        out_shape=(jax.ShapeDtypeStruct((N, 1), jnp.float32),
                   jax.ShapeDtypeStruct((N + 8, HID), jnp.float32)),
    )(degp, x, w1)


def _combine_body(p_ref, xwp_ref, dinv_ref, b_ref, wn_ref, x_ref, xwn_ref):
    dinv = dinv_ref[...]
    nout = xwn_ref.shape[1]
    xc = jnp.tanh((p_ref[0] + p_ref[1] + xwp_ref[0:N, :]) * dinv + b_ref[...])
    x_ref[...] = xc
    xwn_ref[0:N, :] = (xc @ wn_ref[...]) * dinv
    xwn_ref[N:N + 8, :] = jnp.zeros((8, nout), jnp.float32)


def _tc_combine(p, xwp, dinv, b, wnext, nout):
    return pl.pallas_call(
        _combine_body,
        out_shape=(jax.ShapeDtypeStruct((N, HID), jnp.float32),
                   jax.ShapeDtypeStruct((N + 8, nout), jnp.float32)),
    )(p, xwp, dinv, b, wnext)


def _final_body(p4_ref, xw4_ref, dinv_ref, b4_ref, x4_ref):
    agg = jnp.sum(p4_ref[...], axis=0)[:, None]
    x4_ref[...] = jnp.tanh((agg + xw4_ref[0:N, :]) * dinv_ref[...] + b4_ref[0])


def _tc_final(p4, xw4p, dinv, b4):
    return pl.pallas_call(
        _final_body,
        out_shape=jax.ShapeDtypeStruct((N, 1), jnp.float32),
    )(p4, xw4p, dinv, b4)


def _head_body(r1_ref, r2_ref, r3_ref, v_ref, c5w_ref, c5b_ref,
               c6w_ref, c6b_ref, f1w_ref, f1b_ref, f2w_ref, f2b_ref, o_ref):
    p97 = jnp.concatenate(
        [r1_ref[...], r2_ref[...], r3_ref[...], v_ref[...]], axis=1)
    y = jnp.maximum(p97 @ c5w_ref[...] + c5b_ref[...], 0.0)
    y = jnp.max(y.reshape(G * K // 2, 2, 16), axis=1)
    y = y.reshape(G, K // 2, 16)
    cols = [y[:, dt:dt + 11, :] for dt in range(5)]
    z = jnp.concatenate(cols, axis=2).reshape(G * 11, 80)
    h2 = jnp.maximum(z @ c6w_ref[...] + c6b_ref[...], 0.0)
    h3 = h2.reshape(G, 11, 32)
    acc = jnp.zeros((G, 128), jnp.float32)
    for t in range(11):
        acc = acc + h3[:, t, :] @ f1w_ref[t]
    h = jnp.maximum(acc + f1b_ref[...], 0.0)
    zz = h @ f2w_ref[...] + f2b_ref[...]
    m = jnp.max(zz, axis=-1, keepdims=True)
    e = jnp.exp(zz - m)
    o_ref[...] = (zz - m) - jnp.log(jnp.sum(e, axis=-1, keepdims=True))


def _tc_head(r1, r2, r3, v, c5wT, c5b, c6wT, c6b, f1w3, f1b, f2w, f2b):
    return pl.pallas_call(
        _head_body,
        out_shape=jax.ShapeDtypeStruct((G, 10), jnp.float32),
    )(r1, r2, r3, v, c5wT, c5b, c6wT, c6b, f1w3, f1b, f2w, f2b)


def kernel(x, edge_index, batch, W1, b1, W2, b2, W3, b3, W4, b4,
           c5w, c5b, c6w, c6b, f1w, f1b, f2w, f2b):
    pad_len = NW * EPW - E
    pad_idx = (jnp.arange(pad_len, dtype=jnp.int32) % N)
    src = jnp.concatenate([edge_index[0], pad_idx]).reshape(NW, EPW)
    dst = jnp.concatenate([edge_index[1], pad_idx]).reshape(NW, EPW)
    dst3 = dst.reshape(NW, NCHUNK, EC)
    zeros_n32 = jnp.zeros((N, HID), jnp.float32)

    degp, src2 = _prep_sc(src, dst)
    dinv, xw1p = _tc_prep(degp, x, W1)

    p1 = _agg_sc(xw1p, src2, dst3, zeros_n32)
    x1, xw2p = _tc_combine(p1, xw1p, dinv, b1, W2, HID)
    p2 = _agg_sc(xw2p, src2, dst3, zeros_n32)
    x2, xw3p = _tc_combine(p2, xw2p, dinv, b2, W3, HID)
    p3 = _agg_sc(xw3p, src2, dst3, zeros_n32)
    x3, xw4p = _tc_combine(p3, xw3p, dinv, b3, W4, 1)
    p4 = _agg1ch_sc(xw4p.reshape(N + 8), src2, dst)
    x4 = _tc_final(p4, xw4p, dinv, b4)

    r1, r2, r3, v = _pool_sc(x4.reshape(N), batch, x1, x2, x3)
    v = v.reshape(G * K, 1)

    c5wT = c5w[:, 0, :].T
    c6wT = jnp.transpose(c6w, (2, 1, 0)).reshape(80, 32)
    f1w3 = f1w.reshape(32, 11, 128).transpose(1, 0, 2)

    return _tc_head(r1, r2, r3, v, c5wT, c5b, c6wT, c6b, f1w3, f1b, f2w, f2b)


# speedup vs baseline: 1.6288x; 1.6288x over previous
import functools
import numpy as np
import jax
import jax.numpy as jnp
from jax import lax
from jax.experimental import pallas as pl
from jax.experimental.pallas import tpu as pltpu
from jax.experimental.pallas import tpu_sc as plsc

N = 10000
E = 320000
G = 100
K = 30
HID = 32

NC = 2
NS = 16
NW = NC * NS
L = 16

EPW = E // NW
EC = 80
NCHUNK = EPW // EC
RPT = N // NS
RPT8 = 632
NB = 5

_mesh = plsc.VectorSubcoreMesh(core_axis_name="c", subcore_axis_name="s")
_sc_params = pltpu.CompilerParams(needs_layout_passes=False,
                                  use_tc_tiling_on_sc=False)


def _wid():
    return lax.axis_index("s") * NC + lax.axis_index("c")


@functools.partial(
    pl.kernel, mesh=_mesh, compiler_params=_sc_params,
    out_type=(jax.ShapeDtypeStruct((NW, N), jnp.float32),
              jax.ShapeDtypeStruct((NW, EPW), jnp.int32)),
    scratch_types=[
        pltpu.VMEM((N,), jnp.float32),
        pltpu.VMEM((EPW,), jnp.int32),
        pltpu.VMEM((EPW,), jnp.int32),
        pltpu.VMEM((EPW,), jnp.int32),
    ],
)
def _prep_sc(src_hbm, dst_hbm, out_hbm, src2_hbm, deg_v, src_v, dst_v, s2_v):
    w = _wid()
    pltpu.sync_copy(src_hbm.at[w], src_v)
    pltpu.sync_copy(dst_hbm.at[w], dst_v)
    z16 = jnp.zeros((L,), jnp.float32)

    def zero_body(j, _):
        deg_v[pl.ds(j * L, L)] = z16
        return 0
    lax.fori_loop(0, N // L, zero_body, 0)

    def step(i, _):
        s16 = src_v[pl.ds(i * L, L)]
        d16 = dst_v[pl.ds(i * L, L)]
        loop = s16 == d16
        m16 = jnp.where(loop, 0.0, 1.0).astype(jnp.float32)
        s2_v[pl.ds(i * L, L)] = jnp.where(loop, jnp.int32(N), s16)
        plsc.addupdate_scatter(deg_v, [d16], m16)
        return 0
    lax.fori_loop(0, EPW // L, step, 0)
    pltpu.sync_copy(deg_v, out_hbm.at[w])
    pltpu.sync_copy(s2_v, src2_hbm.at[w])


@functools.partial(
    pl.kernel, mesh=_mesh, compiler_params=_sc_params,
    out_type=jax.ShapeDtypeStruct((NC, N, HID), jnp.float32),
    scratch_types=[
        pltpu.VMEM_SHARED((N, HID), jnp.float32),
        pltpu.VMEM((EPW,), jnp.int32),
        pltpu.VMEM((NCHUNK, EC), jnp.int32),
        pltpu.VMEM((NB, EC, HID), jnp.float32),
        [pltpu.SemaphoreType.DMA] * NB,
        [pltpu.SemaphoreType.DMA] * NB,
    ],
)
def _agg_sc(xw_hbm, src_hbm, dst_hbm, zeros_hbm, out_hbm,
            agg_sp, src_v, dst_v, gbuf, gsem, ssem):
    c = lax.axis_index("c")
    s = lax.axis_index("s")
    w = _wid()
    rb = jnp.minimum(s * RPT8, N - RPT8)
    dz = pltpu.async_copy(zeros_hbm.at[pl.ds(rb, RPT8)],
                          agg_sp.at[pl.ds(rb, RPT8)], gsem[0])
    d1 = pltpu.async_copy(src_hbm.at[w], src_v, gsem[1])
    d2 = pltpu.async_copy(dst_hbm.at[w], dst_v, gsem[2])
    dz.wait(); d1.wait(); d2.wait()
    plsc.subcore_barrier()

    for b in range(NB):
        pltpu.async_copy(xw_hbm.at[src_v.at[pl.ds(b * EC, EC)]],
                         gbuf.at[b], gsem[b])

    def chunk(g, _):
        for b in range(NB):
            gg = g * NB + b
            pltpu.make_async_copy(xw_hbm.at[src_v.at[pl.ds(0, EC)]],
                                  gbuf.at[b], gsem[b]).wait()
            pltpu.async_copy(gbuf.at[b], agg_sp.at[dst_v.at[gg]], ssem[b],
                             add=True)

            @pl.when(gg + NB < NCHUNK)
            def _():
                pltpu.make_async_copy(gbuf.at[b], agg_sp.at[dst_v.at[gg]],
                                      ssem[b]).wait()
                pltpu.async_copy(
                    xw_hbm.at[src_v.at[pl.ds((gg + NB) * EC, EC)]],
                    gbuf.at[b], gsem[b])
        return 0
    lax.fori_loop(0, NCHUNK // NB, chunk, 0)
    for b in range(NB):
        pltpu.make_async_copy(gbuf.at[b], agg_sp.at[dst_v.at[0]],
                              ssem[b]).wait()
    plsc.subcore_barrier()
    pltpu.sync_copy(agg_sp.at[pl.ds(rb, RPT8)],
                    out_hbm.at[c, pl.ds(rb, RPT8)])


@functools.partial(
    pl.kernel, mesh=_mesh, compiler_params=_sc_params,
    out_type=jax.ShapeDtypeStruct((NW, N), jnp.float32),
    scratch_types=[
        pltpu.VMEM((N + 8,), jnp.float32),
        pltpu.VMEM((N,), jnp.float32),
        pltpu.VMEM((EPW,), jnp.int32),
        pltpu.VMEM((EPW,), jnp.int32),
    ],
)
def _agg1ch_sc(xw_hbm, src_hbm, dst_hbm, out_hbm, xw_v, acc_v, src_v, dst_v):
    w = _wid()
    pltpu.sync_copy(xw_hbm, xw_v)
    pltpu.sync_copy(src_hbm.at[w], src_v)
    pltpu.sync_copy(dst_hbm.at[w], dst_v)
    z16 = jnp.zeros((L,), jnp.float32)

    def zero_body(j, _):
        acc_v[pl.ds(j * L, L)] = z16
        return 0
    lax.fori_loop(0, N // L, zero_body, 0)

    def step(i, _):
        s16 = src_v[pl.ds(i * L, L)]
        d16 = dst_v[pl.ds(i * L, L)]
        v16 = plsc.load_gather(xw_v, [s16])
        plsc.addupdate_scatter(acc_v, [d16], v16)
        return 0
    lax.fori_loop(0, EPW // L, step, 0)
    pltpu.sync_copy(acc_v, out_hbm.at[w])


GPW = 4
AW = G // GPW
SPW = GPW * K

_NEG = np.float32(-3.4e38)


def _iota():
    return lax.iota(jnp.int32, L)


def _lane_i32(v16, lane):
    return jnp.max(jnp.where(_iota() == lane, v16, jnp.int32(-2**31)))


@functools.partial(
    pl.kernel, mesh=_mesh, compiler_params=_sc_params,
    out_type=(jax.ShapeDtypeStruct((G * K, HID), jnp.float32),
              jax.ShapeDtypeStruct((G * K, HID), jnp.float32),
              jax.ShapeDtypeStruct((G * K, HID), jnp.float32),
              jax.ShapeDtypeStruct((G * K,), jnp.float32)),
    scratch_types=[
        pltpu.VMEM((N,), jnp.float32),
        pltpu.VMEM((N,), jnp.int32),
        pltpu.VMEM((128,), jnp.int32),
        pltpu.VMEM((128,), jnp.int32),
        pltpu.VMEM((128,), jnp.int32),
        pltpu.VMEM((128,), jnp.float32),
        pltpu.VMEM((128, HID), jnp.float32),
        pltpu.VMEM((128, HID), jnp.float32),
        pltpu.VMEM((128, HID), jnp.float32),
        pltpu.SemaphoreType.DMA,
    ],
)
def _pool_sc(vals_hbm, batch_hbm, x1_hbm, x2_hbm, x3_hbm,
             o1_hbm, o2_hbm, o3_hbm, ov_hbm,
             vals_v, batch_v, cnt_v, starts_v, idx_v, valb_v,
             r1_v, r2_v, r3_v, sem):
    w = _wid()

    def body():
        pltpu.sync_copy(vals_hbm, vals_v)
        pltpu.sync_copy(batch_hbm, batch_v)
        z16i = jnp.zeros((L,), jnp.int32)
        z16f = jnp.zeros((L,), jnp.float32)
        one16 = jnp.ones((L,), jnp.int32)
        for j in range(128 // L):
            cnt_v[pl.ds(j * L, L)] = z16i
            idx_v[pl.ds(j * L, L)] = z16i
            valb_v[pl.ds(j * L, L)] = z16f

        def cnt_body(t, _):
            b16 = batch_v[pl.ds(t * L, L)]
            plsc.addupdate_scatter(cnt_v, [b16], one16)
            return 0
        lax.fori_loop(0, N // L, cnt_body, 0)

        carry = jnp.int32(0)
        for j in range(128 // L):
            c16 = cnt_v[pl.ds(j * L, L)]
            inc = plsc.cumsum(c16)
            starts_v[pl.ds(j * L, L)] = inc - c16 + carry
            carry = carry + jnp.sum(c16)

        for j in range(GPW):
            g = w * GPW + j
            gb = (g // L) * L
            s16 = starts_v[pl.ds(gb, L)]
            c16 = cnt_v[pl.ds(gb, L)]
            s = _lane_i32(s16, g - gb)
            c = _lane_i32(c16, g - gb)
            m = jnp.minimum(jnp.int32(K), c)
            b0 = (s // L) * L
            nch = (s + c - b0 + (L - 1)) // L

            def k_body(k, _):
                def t_body(t, bc):
                    bv, bi = bc
                    off = b0 + t * L
                    v = vals_v[pl.ds(off, L)]
                    gi = off + _iota()
                    ok = (gi >= s) & (gi < s + c)
                    vm = jnp.where(ok, v, _NEG)
                    cm = jnp.max(vm)
                    gmin = jnp.min(jnp.where(vm == cm, gi, jnp.int32(2**30)))
                    better = cm > bv
                    return (jnp.where(better, cm, bv),
                            jnp.where(better, gmin, bi))
                bv, bi = lax.fori_loop(0, nch, t_body,
                                       (jnp.float32(-2.0e38), jnp.int32(0)))
                slot16 = jnp.full((L,), j * K + k, jnp.int32)
                bi16 = jnp.full((L,), bi, jnp.int32)
                lane0 = _iota() == 0
                plsc.store_scatter(idx_v, [slot16], bi16, mask=lane0)
                plsc.store_scatter(valb_v, [slot16],
                                   jnp.full((L,), bv, jnp.float32), mask=lane0)
                plsc.store_scatter(vals_v, [bi16],
                                   jnp.full((L,), _NEG, jnp.float32), mask=lane0)
                return 0
            lax.fori_loop(0, m, k_body, 0)

        d1 = pltpu.async_copy(x1_hbm.at[idx_v], r1_v, sem)
        d2 = pltpu.async_copy(x2_hbm.at[idx_v], r2_v, sem)
        d3 = pltpu.async_copy(x3_hbm.at[idx_v], r3_v, sem)
        d1.wait(); d2.wait(); d3.wait()

        for j in range(GPW):
            g = w * GPW + j
            gb = (g // L) * L
            c16 = cnt_v[pl.ds(gb, L)]
            m = jnp.minimum(jnp.int32(K), _lane_i32(c16, g - gb))

            def z_body(k, _):
                slot16 = jnp.full((L,), j * K + k, jnp.int32)
                for rv in (r1_v, r2_v, r3_v):
                    plsc.store_scatter(rv, [slot16, _iota()], z16f)
                    plsc.store_scatter(rv, [slot16, _iota() + L], z16f)
                return 0
            lax.fori_loop(m, K, z_body, 0)

        pltpu.sync_copy(r1_v.at[pl.ds(0, SPW)], o1_hbm.at[pl.ds(w * SPW, SPW)])
        pltpu.sync_copy(r2_v.at[pl.ds(0, SPW)], o2_hbm.at[pl.ds(w * SPW, SPW)])
        pltpu.sync_copy(r3_v.at[pl.ds(0, SPW)], o3_hbm.at[pl.ds(w * SPW, SPW)])
        pltpu.sync_copy(valb_v.at[pl.ds(0, SPW)], ov_hbm.at[pl.ds(w * SPW, SPW)])

    pl.when(w < AW)(body)


def _prep_body(degp_ref, x_ref, w1_ref, dinv_ref, xw1_ref):
    deg = jnp.sum(degp_ref[...], axis=0) + 1.0
    dinv = lax.rsqrt(deg)[:, None]
    dinv_ref[...] = dinv
    xw1_ref[0:N, :] = (x_ref[...] @ w1_ref[...]) * dinv
    xw1_ref[N:N + 8, :] = jnp.zeros((8, HID), jnp.float32)


def _tc_prep(degp, x, w1):
    return pl.pallas_call(
        _prep_body,
        out_shape=(jax.ShapeDtypeStruct((N, 1), jnp.float32),
                   jax.ShapeDtypeStruct((N + 8, HID), jnp.float32)),
    )(degp, x, w1)


def _combine_body(p_ref, xwp_ref, dinv_ref, b_ref, wn_ref, x_ref, xwn_ref):
    dinv = dinv_ref[...]
    nout = xwn_ref.shape[1]
    xc = jnp.tanh((p_ref[0] + p_ref[1] + xwp_ref[0:N, :]) * dinv + b_ref[...])
    x_ref[...] = xc
    xwn_ref[0:N, :] = (xc @ wn_ref[...]) * dinv
    xwn_ref[N:N + 8, :] = jnp.zeros((8, nout), jnp.float32)


def _tc_combine(p, xwp, dinv, b, wnext, nout):
    return pl.pallas_call(
        _combine_body,
        out_shape=(jax.ShapeDtypeStruct((N, HID), jnp.float32),
                   jax.ShapeDtypeStruct((N + 8, nout), jnp.float32)),
    )(p, xwp, dinv, b, wnext)


def _final_body(p4_ref, xw4_ref, dinv_ref, b4_ref, x4_ref):
    agg = jnp.sum(p4_ref[...], axis=0)[:, None]
    x4_ref[...] = jnp.tanh((agg + xw4_ref[0:N, :]) * dinv_ref[...] + b4_ref[0])


def _tc_final(p4, xw4p, dinv, b4):
    return pl.pallas_call(
        _final_body,
        out_shape=jax.ShapeDtypeStruct((N, 1), jnp.float32),
    )(p4, xw4p, dinv, b4)


def _head_body(r1_ref, r2_ref, r3_ref, v_ref, c5w_ref, c5b_ref,
               c6w_ref, c6b_ref, f1w_ref, f1b_ref, f2w_ref, f2b_ref, o_ref):
    p97 = jnp.concatenate(
        [r1_ref[...], r2_ref[...], r3_ref[...], v_ref[...]], axis=1)
    y = jnp.maximum(p97 @ c5w_ref[...] + c5b_ref[...], 0.0)
    y = jnp.max(y.reshape(G * K // 2, 2, 16), axis=1)
    y = y.reshape(G, K // 2, 16)
    cols = [y[:, dt:dt + 11, :] for dt in range(5)]
    z = jnp.concatenate(cols, axis=2).reshape(G * 11, 80)
    h2 = jnp.maximum(z @ c6w_ref[...] + c6b_ref[...], 0.0)
    h3 = h2.reshape(G, 11, 32)
    acc = jnp.zeros((G, 128), jnp.float32)
    for t in range(11):
        acc = acc + h3[:, t, :] @ f1w_ref[t]
    h = jnp.maximum(acc + f1b_ref[...], 0.0)
    zz = h @ f2w_ref[...] + f2b_ref[...]
    m = jnp.max(zz, axis=-1, keepdims=True)
    e = jnp.exp(zz - m)
    o_ref[...] = (zz - m) - jnp.log(jnp.sum(e, axis=-1, keepdims=True))


def _tc_head(r1, r2, r3, v, c5wT, c5b, c6wT, c6b, f1w3, f1b, f2w, f2b):
    return pl.pallas_call(
        _head_body,
        out_shape=jax.ShapeDtypeStruct((G, 10), jnp.float32),
    )(r1, r2, r3, v, c5wT, c5b, c6wT, c6b, f1w3, f1b, f2w, f2b)


def kernel(x, edge_index, batch, W1, b1, W2, b2, W3, b3, W4, b4,
           c5w, c5b, c6w, c6b, f1w, f1b, f2w, f2b):
    src = edge_index[0].reshape(NW, EPW)
    dst = edge_index[1].reshape(NW, EPW)
    dst3 = dst.reshape(NW, NCHUNK, EC)
    zeros_n32 = jnp.zeros((N, HID), jnp.float32)

    degp, src2 = _prep_sc(src, dst)
    dinv, xw1p = _tc_prep(degp, x, W1)

    p1 = _agg_sc(xw1p, src2, dst3, zeros_n32)
    x1, xw2p = _tc_combine(p1, xw1p, dinv, b1, W2, HID)
    p2 = _agg_sc(xw2p, src2, dst3, zeros_n32)
    x2, xw3p = _tc_combine(p2, xw2p, dinv, b2, W3, HID)
    p3 = _agg_sc(xw3p, src2, dst3, zeros_n32)
    x3, xw4p = _tc_combine(p3, xw3p, dinv, b3, W4, 1)
    p4 = _agg1ch_sc(xw4p.reshape(N + 8), src2, dst)
    x4 = _tc_final(p4, xw4p, dinv, b4)

    r1, r2, r3, v = _pool_sc(x4.reshape(N), batch, x1, x2, x3)
    v = v.reshape(G * K, 1)

    c5wT = c5w[:, 0, :].T
    c6wT = jnp.transpose(c6w, (2, 1, 0)).reshape(80, 32)
    f1w3 = f1w.reshape(32, 11, 128).transpose(1, 0, 2)

    return _tc_head(r1, r2, r3, v, c5wT, c5b, c6wT, c6b, f1w3, f1b, f2w, f2b)
